# baseline (speedup 1.0000x reference)
import functools

import jax
import jax.numpy as jnp
from jax import lax
from jax.experimental import pallas as pl
from jax.experimental.pallas import tpu as pltpu

_TABLE_ROWS = 1024
_DIM = 768
_B = 16 * 1024
_BB = 512


def _tc_body(idx_ref, tab_ref, out_ref):
    idx_col = idx_ref[...]
    vids = lax.broadcasted_iota(jnp.int32, (_BB, _TABLE_ROWS), 1)
    onehot = (idx_col == vids).astype(jnp.bfloat16)
    out_ref[...] = lax.dot_general(
        onehot, tab_ref[...],
        (((1,), (0,)), ((), ())),
        preferred_element_type=jnp.float32)


@jax.jit
def _tc_lookup(embeddings, idx_flat):
    tab16 = embeddings.astype(jnp.bfloat16)
    idx_col = idx_flat.reshape(_B, 1)
    return pl.pallas_call(
        _tc_body,
        grid=(_B // _BB,),
        in_specs=[
            pl.BlockSpec((_BB, 1), lambda i: (i, 0)),
            pl.BlockSpec((_TABLE_ROWS, _DIM), lambda i: (0, 0)),
        ],
        out_specs=pl.BlockSpec((_BB, _DIM), lambda i: (i, 0)),
        out_shape=jax.ShapeDtypeStruct((_B, _DIM), jnp.float32),
    )(idx_col, tab16)


def kernel(patch_index, embeddings):
    idx_flat = patch_index.reshape(-1)
    out = _tc_lookup(embeddings, idx_flat)
    return out.reshape(patch_index.shape + (embeddings.shape[1],))


# speedup vs baseline: 1.1950x; 1.1950x over previous
import functools

import jax
import jax.numpy as jnp
from jax import lax
from jax.experimental import pallas as pl
from jax.experimental.pallas import tpu as pltpu

_TABLE_ROWS = 1024
_DIM = 768
_B = 16 * 1024
_BB = 2048


def _tc_body(idx_ref, tab_ref, out_ref):
    idx_col = idx_ref[...]
    vids = lax.broadcasted_iota(jnp.int32, (_BB, _TABLE_ROWS), 1)
    onehot = (idx_col == vids).astype(jnp.bfloat16)
    out_ref[...] = lax.dot_general(
        onehot, tab_ref[...],
        (((1,), (0,)), ((), ())),
        preferred_element_type=jnp.float32)


@jax.jit
def _tc_lookup(embeddings, idx_flat):
    tab16 = embeddings.astype(jnp.bfloat16)
    idx_col = idx_flat.reshape(_B, 1)
    return pl.pallas_call(
        _tc_body,
        grid=(_B // _BB,),
        in_specs=[
            pl.BlockSpec((_BB, 1), lambda i: (i, 0)),
            pl.BlockSpec((_TABLE_ROWS, _DIM), lambda i: (0, 0)),
        ],
        out_specs=pl.BlockSpec((_BB, _DIM), lambda i: (i, 0)),
        out_shape=jax.ShapeDtypeStruct((_B, _DIM), jnp.float32),
    )(idx_col, tab16)


def kernel(patch_index, embeddings):
    idx_flat = patch_index.reshape(-1)
    out = _tc_lookup(embeddings, idx_flat)
    return out.reshape(patch_index.shape + (embeddings.shape[1],))
